# Initial kernel scaffold; baseline (speedup 1.0000x reference)
import jax
import jax.numpy as jnp
from jax.experimental import pallas as pl


def kernel(X, Mu, Var, W, S, lin_l_w, lin_l_b, lin_r_w, lin_r_b, att, edge_index):
    raise NotImplementedError("write your pallas kernel here")


# speedup vs baseline: 17.3720x; 17.3720x over previous
import functools

import jax
import jax.numpy as jnp
from jax import lax
from jax.experimental import pallas as pl
from jax.experimental.pallas import tpu as pltpu
from jax.experimental.pallas import tpu_sc as plsc

N = 10000
C = 20
G = 128
E = 320000
H = 8

DW = 32
NW = 32
EPT = E // NW
EB = 80
NCHUNK = EPT // EB
GPC = EB // 16
NPAD = 10240

BN = 1000
GRID = N // BN


def _dense_body(x_ref, mu_ref, var_ref, w_ref, s_ref, llw_ref, lrw_ref,
                llb_ref, lrb_ref, p_ref, src_ref, dst_ref, ll_ref):
    i = pl.program_id(0)
    wb = w_ref[...]
    m = jnp.max(wb, axis=1, keepdims=True)
    ew = jnp.exp(wb - m)
    p = ew / jnp.sum(ew, axis=1, keepdims=True)
    lq = jnp.log(p + 1e-8)

    mu = mu_ref[...]
    iv = 1.0 / var_ref[...]
    mv = mu * iv
    dvec = jnp.sum(mu * mv, axis=1)

    xb = x_ref[...]
    a = lax.dot_general(xb * xb, iv, (((1,), (1,)), ((), ())),
                        preferred_element_type=jnp.float32)
    b = lax.dot_general(xb, mv, (((1,), (1,)), ((), ())),
                        preferred_element_type=jnp.float32)
    s = s_ref[...]
    f = -0.5 * (a - 2.0 * s * b + (s * s) * dvec[None, :])
    part = jnp.sum(p * f) * (1.0 / N)

    @pl.when(i == 0)
    def _():
        ll_ref[...] = jnp.zeros((1, 1), jnp.float32)

    ll_ref[...] += jnp.reshape(part, (1, 1))

    xl = lax.dot_general(xb, llw_ref[...], (((1,), (1,)), ((), ())),
                         preferred_element_type=jnp.float32) + llb_ref[...]
    xr = lax.dot_general(xb, lrw_ref[...], (((1,), (1,)), ((), ())),
                         preferred_element_type=jnp.float32) + lrb_ref[...]
    zpad = jnp.zeros((BN, DW - H - C), jnp.float32)
    src_ref[...] = jnp.concatenate([xl, p, zpad], axis=1)
    dst_ref[...] = jnp.concatenate([xr, lq, zpad], axis=1)
    p_ref[...] = p


_dense_call = pl.pallas_call(
    _dense_body,
    grid=(GRID,),
    in_specs=[
        pl.BlockSpec((BN, G), lambda i: (i, 0)),
        pl.BlockSpec((C, G), lambda i: (0, 0)),
        pl.BlockSpec((C, G), lambda i: (0, 0)),
        pl.BlockSpec((BN, C), lambda i: (i, 0)),
        pl.BlockSpec((BN, 1), lambda i: (i, 0)),
        pl.BlockSpec((H, G), lambda i: (0, 0)),
        pl.BlockSpec((H, G), lambda i: (0, 0)),
        pl.BlockSpec((1, H), lambda i: (0, 0)),
        pl.BlockSpec((1, H), lambda i: (0, 0)),
    ],
    out_specs=[
        pl.BlockSpec((BN, C), lambda i: (i, 0)),
        pl.BlockSpec((BN, DW), lambda i: (i, 0)),
        pl.BlockSpec((BN, DW), lambda i: (i, 0)),
        pl.BlockSpec((1, 1), lambda i: (0, 0)),
    ],
    out_shape=[
        jax.ShapeDtypeStruct((N, C), jnp.float32),
        jax.ShapeDtypeStruct((N, DW), jnp.float32),
        jax.ShapeDtypeStruct((N, DW), jnp.float32),
        jax.ShapeDtypeStruct((1, 1), jnp.float32),
    ],
)


_sc_mesh = plsc.VectorSubcoreMesh(core_axis_name="c", subcore_axis_name="s")


@functools.partial(
    pl.kernel,
    mesh=_sc_mesh,
    compiler_params=pltpu.CompilerParams(
        needs_layout_passes=False, use_tc_tiling_on_sc=False),
    out_type=(
        jax.ShapeDtypeStruct((NW, NPAD), jnp.float32),
        jax.ShapeDtypeStruct((NW, NPAD), jnp.float32),
    ),
    scratch_types=[
        pltpu.VMEM((EPT,), jnp.int32),
        pltpu.VMEM((EPT,), jnp.int32),
        pltpu.VMEM((EB, DW), jnp.float32),
        pltpu.VMEM((EB, DW), jnp.float32),
        pltpu.VMEM((EB, DW), jnp.float32),
        pltpu.VMEM((EB, DW), jnp.float32),
        pltpu.VMEM((NPAD,), jnp.float32),
        pltpu.VMEM((NPAD,), jnp.float32),
        pltpu.VMEM((16,), jnp.float32),
        pltpu.SemaphoreType.DMA,
        pltpu.SemaphoreType.DMA,
    ],
)
def _edge_kernel(src_tab, dst_tab, eidx, att16, den_out, t_out,
                 src_idx, dst_idx, bs0, bs1, bd0, bd1, acc_d, acc_t,
                 att_v, sem0, sem1):
    cid = lax.axis_index("c")
    sid = lax.axis_index("s")
    wid = sid * 2 + cid
    ebase = wid * EPT

    pltpu.sync_copy(eidx.at[0, pl.ds(ebase, EPT)], src_idx)
    pltpu.sync_copy(eidx.at[1, pl.ds(ebase, EPT)], dst_idx)
    pltpu.sync_copy(att16, att_v)

    def _zero(k, carry):
        acc_d[pl.ds(k * 16, 16)] = jnp.zeros((16,), jnp.float32)
        acc_t[pl.ds(k * 16, 16)] = jnp.zeros((16,), jnp.float32)
        return carry

    lax.fori_loop(0, NPAD // 16, _zero, 0)

    attvec = att_v[...]
    att_s = [attvec[j] for j in range(H)]
    iota16 = lax.iota(jnp.int32, 16)

    def fire(c, bs, bd, sem):
        pltpu.async_copy(src_tab.at[src_idx.at[pl.ds(c * EB, EB)]], bs, sem)
        pltpu.async_copy(dst_tab.at[dst_idx.at[pl.ds(c * EB, EB)]], bd, sem)

    def drain(bs, bd, sem):
        pltpu.make_async_copy(src_tab.at[pl.ds(0, EB)], bs, sem).wait()
        pltpu.make_async_copy(dst_tab.at[pl.ds(0, EB)], bd, sem).wait()

    def compute(c, bs, bd):
        for g in range(GPC):
            rows = iota16 + (g * 16)
            dstv = dst_idx[pl.ds(c * EB + g * 16, 16)]
            score = jnp.zeros((16,), jnp.float32)
            for j in range(H):
                cols = jnp.full((16,), j, jnp.int32)
                av = plsc.load_gather(bs, [rows, cols])
                bv = plsc.load_gather(bd, [rows, cols])
                z = av + bv
                z = jnp.where(z >= 0.0, z, 0.2 * z)
                score = score + att_s[j] * z
            q = jnp.zeros((16,), jnp.float32)
            for j in range(H, H + C):
                cols = jnp.full((16,), j, jnp.int32)
                av = plsc.load_gather(bs, [rows, cols])
                bv = plsc.load_gather(bd, [rows, cols])
                q = q + av * bv
            w = jnp.exp(score)
            plsc.addupdate_scatter(acc_d, [dstv], w)
            plsc.addupdate_scatter(acc_t, [dstv], w * q)

    fire(0, bs0, bd0, sem0)

    def body(cc, carry):
        c0 = cc * 2
        drain(bs0, bd0, sem0)
        fire(c0 + 1, bs1, bd1, sem1)
        compute(c0, bs0, bd0)
        drain(bs1, bd1, sem1)
        fire(c0 + 2, bs0, bd0, sem0)
        compute(c0 + 1, bs1, bd1)
        return carry

    lax.fori_loop(0, (NCHUNK - 1) // 2, body, 0)

    drain(bs0, bd0, sem0)
    compute(NCHUNK - 1, bs0, bd0)

    pltpu.sync_copy(acc_d, den_out.at[wid])
    pltpu.sync_copy(acc_t, t_out.at[wid])


def _fin_body(d_ref, t_ref, ce_ref):
    d = jnp.sum(d_ref[...], axis=0)
    t = jnp.sum(t_ref[...], axis=0)
    safe = jnp.where(d > 0.0, d, 1.0)
    ce = -jnp.sum(jnp.where(d > 0.0, t / safe, 0.0)) * (1.0 / N)
    ce_ref[...] = jnp.reshape(ce, (1, 1))


_fin_call = pl.pallas_call(
    _fin_body,
    in_specs=[
        pl.BlockSpec((NW, NPAD), lambda: (0, 0)),
        pl.BlockSpec((NW, NPAD), lambda: (0, 0)),
    ],
    out_specs=pl.BlockSpec((1, 1), lambda: (0, 0)),
    out_shape=jax.ShapeDtypeStruct((1, 1), jnp.float32),
)


def kernel(X, Mu, Var, W, S, lin_l_w, lin_l_b, lin_r_w, lin_r_b, att, edge_index):
    p, src_tab, dst_tab, ll = _dense_call(
        X, Mu, Var, W, S, lin_l_w, lin_r_w,
        lin_l_b.reshape(1, H), lin_r_b.reshape(1, H))
    att16 = jnp.pad(att, (0, 16 - H))
    den, t = _edge_kernel(src_tab, dst_tab, edge_index, att16)
    ce = _fin_call(den, t)
    return (ll[0, 0], ce[0, 0], p)


# speedup vs baseline: 35.4014x; 2.0378x over previous
import functools

import jax
import jax.numpy as jnp
from jax import lax
from jax.experimental import pallas as pl
from jax.experimental.pallas import tpu as pltpu
from jax.experimental.pallas import tpu_sc as plsc

N = 10000
C = 20
G = 128
E = 320000
H = 8

DW = 32
NW = 32
EPT = E // NW
EB = 80
NCHUNK = EPT // EB
GPC = EB // 16
NPAD = 10240

BN = 1000
GRID = N // BN


def _dense_body(x_ref, mu_ref, var_ref, w_ref, s_ref, llw_ref, lrw_ref,
                llb_ref, lrb_ref, p_ref, src_ref, dst_ref, ll_ref):
    i = pl.program_id(0)
    wb = w_ref[...]
    m = jnp.max(wb, axis=1, keepdims=True)
    ew = jnp.exp(wb - m)
    p = ew / jnp.sum(ew, axis=1, keepdims=True)
    lq = jnp.log(p + 1e-8)

    mu = mu_ref[...]
    iv = 1.0 / var_ref[...]
    mv = mu * iv
    dvec = jnp.sum(mu * mv, axis=1)

    xb = x_ref[...]
    a = lax.dot_general(xb * xb, iv, (((1,), (1,)), ((), ())),
                        preferred_element_type=jnp.float32)
    b = lax.dot_general(xb, mv, (((1,), (1,)), ((), ())),
                        preferred_element_type=jnp.float32)
    s = s_ref[...]
    f = -0.5 * (a - 2.0 * s * b + (s * s) * dvec[None, :])
    part = jnp.sum(p * f) * (1.0 / N)

    @pl.when(i == 0)
    def _():
        ll_ref[...] = jnp.zeros((1, 1), jnp.float32)

    ll_ref[...] += jnp.reshape(part, (1, 1))

    xl = lax.dot_general(xb, llw_ref[...], (((1,), (1,)), ((), ())),
                         preferred_element_type=jnp.float32) + llb_ref[...]
    xr = lax.dot_general(xb, lrw_ref[...], (((1,), (1,)), ((), ())),
                         preferred_element_type=jnp.float32) + lrb_ref[...]
    zpad = jnp.zeros((BN, DW - H - C), jnp.float32)
    src_ref[...] = jnp.concatenate([xl, p, zpad], axis=1)
    dst_ref[...] = jnp.concatenate([xr, lq, zpad], axis=1)
    p_ref[...] = p


_dense_call = pl.pallas_call(
    _dense_body,
    grid=(GRID,),
    in_specs=[
        pl.BlockSpec((BN, G), lambda i: (i, 0)),
        pl.BlockSpec((C, G), lambda i: (0, 0)),
        pl.BlockSpec((C, G), lambda i: (0, 0)),
        pl.BlockSpec((BN, C), lambda i: (i, 0)),
        pl.BlockSpec((BN, 1), lambda i: (i, 0)),
        pl.BlockSpec((H, G), lambda i: (0, 0)),
        pl.BlockSpec((H, G), lambda i: (0, 0)),
        pl.BlockSpec((1, H), lambda i: (0, 0)),
        pl.BlockSpec((1, H), lambda i: (0, 0)),
    ],
    out_specs=[
        pl.BlockSpec((BN, C), lambda i: (i, 0)),
        pl.BlockSpec((BN, DW), lambda i: (i, 0)),
        pl.BlockSpec((BN, DW), lambda i: (i, 0)),
        pl.BlockSpec((1, 1), lambda i: (0, 0)),
    ],
    out_shape=[
        jax.ShapeDtypeStruct((N, C), jnp.float32),
        jax.ShapeDtypeStruct((N, DW), jnp.float32),
        jax.ShapeDtypeStruct((N, DW), jnp.float32),
        jax.ShapeDtypeStruct((1, 1), jnp.float32),
    ],
)


_sc_mesh = plsc.VectorSubcoreMesh(core_axis_name="c", subcore_axis_name="s")


@functools.partial(
    pl.kernel,
    mesh=_sc_mesh,
    compiler_params=pltpu.CompilerParams(
        needs_layout_passes=False, use_tc_tiling_on_sc=False),
    out_type=(
        jax.ShapeDtypeStruct((NW, NPAD), jnp.float32),
        jax.ShapeDtypeStruct((NW, NPAD), jnp.float32),
    ),
    scratch_types=[
        pltpu.VMEM((EPT,), jnp.int32),
        pltpu.VMEM((EPT,), jnp.int32),
        pltpu.VMEM((EB, DW), jnp.float32),
        pltpu.VMEM((EB, DW), jnp.float32),
        pltpu.VMEM((EB, DW), jnp.float32),
        pltpu.VMEM((EB, DW), jnp.float32),
        pltpu.VMEM((NPAD,), jnp.float32),
        pltpu.VMEM((NPAD,), jnp.float32),
        pltpu.VMEM((128,), jnp.float32),
        pltpu.SemaphoreType.DMA,
        pltpu.SemaphoreType.DMA,
    ],
)
def _edge_kernel(src_tab, dst_tab, eidx, att16, den_out, t_out,
                 src_idx, dst_idx, bs0, bs1, bd0, bd1, acc_d, acc_t,
                 att_v, sem0, sem1):
    cid = lax.axis_index("c")
    sid = lax.axis_index("s")
    wid = sid * 2 + cid
    ebase = wid * EPT

    pltpu.sync_copy(eidx.at[0, pl.ds(ebase, EPT)], src_idx)
    pltpu.sync_copy(eidx.at[1, pl.ds(ebase, EPT)], dst_idx)
    pltpu.sync_copy(att16, att_v)

    def _zero(k, carry):
        acc_d[pl.ds(k * 16, 16)] = jnp.zeros((16,), jnp.float32)
        acc_t[pl.ds(k * 16, 16)] = jnp.zeros((16,), jnp.float32)
        return carry

    lax.fori_loop(0, NPAD // 16, _zero, 0)

    iota16 = lax.iota(jnp.int32, 16)
    att_rot = [att_v[pl.ds(j * 16, 16)] for j in range(H)]
    col8 = [jnp.bitwise_and(iota16 + j, 7) for j in range(H)]
    col20 = []
    for j in range(C):
        cj = iota16 + j
        col20.append(jnp.where(cj >= C, cj - C, cj) + H)

    def fire(c, bs, bd, sem):
        pltpu.async_copy(src_tab.at[src_idx.at[pl.ds(c * EB, EB)]], bs, sem)
        pltpu.async_copy(dst_tab.at[dst_idx.at[pl.ds(c * EB, EB)]], bd, sem)

    def drain(bs, bd, sem):
        pltpu.make_async_copy(src_tab.at[pl.ds(0, EB)], bs, sem).wait()
        pltpu.make_async_copy(dst_tab.at[pl.ds(0, EB)], bd, sem).wait()

    def compute(c, bs, bd):
        for g in range(GPC):
            rows = iota16 + (g * 16)
            dstv = dst_idx[pl.ds(c * EB + g * 16, 16)]
            score = jnp.zeros((16,), jnp.float32)
            for j in range(H):
                av = plsc.load_gather(bs, [rows, col8[j]])
                bv = plsc.load_gather(bd, [rows, col8[j]])
                z = av + bv
                z = jnp.where(z >= 0.0, z, 0.2 * z)
                score = score + att_rot[j] * z
            q = jnp.zeros((16,), jnp.float32)
            for j in range(C):
                av = plsc.load_gather(bs, [rows, col20[j]])
                bv = plsc.load_gather(bd, [rows, col20[j]])
                q = q + av * bv
            w = jnp.exp(score)
            plsc.addupdate_scatter(acc_d, [dstv], w)
            plsc.addupdate_scatter(acc_t, [dstv], w * q)

    fire(0, bs0, bd0, sem0)

    def body(cc, carry):
        c0 = cc * 2
        drain(bs0, bd0, sem0)
        fire(c0 + 1, bs1, bd1, sem1)
        compute(c0, bs0, bd0)
        drain(bs1, bd1, sem1)
        fire(c0 + 2, bs0, bd0, sem0)
        compute(c0 + 1, bs1, bd1)
        return carry

    lax.fori_loop(0, (NCHUNK - 1) // 2, body, 0)

    drain(bs0, bd0, sem0)
    compute(NCHUNK - 1, bs0, bd0)

    pltpu.sync_copy(acc_d, den_out.at[wid])
    pltpu.sync_copy(acc_t, t_out.at[wid])


def _fin_body(d_ref, t_ref, ce_ref):
    d = jnp.sum(d_ref[...], axis=0)
    t = jnp.sum(t_ref[...], axis=0)
    safe = jnp.where(d > 0.0, d, 1.0)
    ce = -jnp.sum(jnp.where(d > 0.0, t / safe, 0.0)) * (1.0 / N)
    ce_ref[...] = jnp.reshape(ce, (1, 1))


_fin_call = pl.pallas_call(
    _fin_body,
    in_specs=[
        pl.BlockSpec((NW, NPAD), lambda: (0, 0)),
        pl.BlockSpec((NW, NPAD), lambda: (0, 0)),
    ],
    out_specs=pl.BlockSpec((1, 1), lambda: (0, 0)),
    out_shape=jax.ShapeDtypeStruct((1, 1), jnp.float32),
)


def kernel(X, Mu, Var, W, S, lin_l_w, lin_l_b, lin_r_w, lin_r_b, att, edge_index):
    p, src_tab, dst_tab, ll = _dense_call(
        X, Mu, Var, W, S, lin_l_w, lin_r_w,
        lin_l_b.reshape(1, H), lin_r_b.reshape(1, H))
    att2 = jnp.concatenate([att, att])
    att_rot = jnp.concatenate([
        lax.dynamic_slice(jnp.concatenate([att2, att2]), (j,), (16,))
        for j in range(H)])
    den, t = _edge_kernel(src_tab, dst_tab, edge_index, att_rot)
    ce = _fin_call(den, t)
    return (ll[0, 0], ce[0, 0], p)
